# Initial kernel scaffold; baseline (speedup 1.0000x reference)
import jax
import jax.numpy as jnp
from jax.experimental import pallas as pl


def kernel(n_feat, edge_index, W0, b0, eps0, W1, b1, eps1, W2, b2, eps2):
    raise NotImplementedError("write your pallas kernel here")


# speedup vs baseline: 1.0403x; 1.0403x over previous
import functools

import jax
import jax.numpy as jnp
from jax.experimental import pallas as pl
from jax.experimental.pallas import tpu as pltpu

N = 10000
E = 320000
D = 128


def _affine_body(h_ref, agg_ref, w_ref, b_ref, eps_ref, o_ref, *, act):
    agg = agg_ref[...]
    agg = jnp.where(jnp.isfinite(agg), agg, 0.0)
    x = (1.0 + eps_ref[0]) * h_ref[...] + agg
    y = jax.lax.dot_general(
        x, w_ref[...],
        dimension_numbers=(((1,), (1,)), ((), ())),
        preferred_element_type=jnp.float32,
    ) + b_ref[...]
    if act:
        y = jnp.where(y >= 0, y, 0.01 * y)
    o_ref[...] = y


def _affine(h, agg, W, b, eps, act):
    return pl.pallas_call(
        functools.partial(_affine_body, act=act),
        out_shape=jax.ShapeDtypeStruct((N, D), jnp.float32),
        in_specs=[
            pl.BlockSpec(memory_space=pltpu.VMEM),
            pl.BlockSpec(memory_space=pltpu.VMEM),
            pl.BlockSpec(memory_space=pltpu.VMEM),
            pl.BlockSpec(memory_space=pltpu.VMEM),
            pl.BlockSpec(memory_space=pltpu.SMEM),
        ],
        out_specs=pl.BlockSpec(memory_space=pltpu.VMEM),
    )(h, agg, W, b.reshape(1, D), eps.reshape(1))


def kernel(n_feat, edge_index, W0, b0, eps0, W1, b1, eps1, W2, b2, eps2):
    src = edge_index[0]
    dst = edge_index[1]
    h = n_feat
    params = ((W0, b0, eps0), (W1, b1, eps1), (W2, b2, eps2))
    for i, (W, b, eps) in enumerate(params):
        agg = jax.ops.segment_max(h[src], dst, num_segments=N)
        h = _affine(h, agg, W, b, eps, act=(i + 1 < len(params)))
    return h
